# Initial kernel scaffold; baseline (speedup 1.0000x reference)
import jax
import jax.numpy as jnp
from jax.experimental import pallas as pl


def kernel(x, edge_index, batch, W1, as1, ad1, b1, W2, as2, ad2, b2, W3, as3, ad3, b3, fc1_W, fc1_b, fc2_W, fc2_b):
    raise NotImplementedError("write your pallas kernel here")


# speedup vs baseline: 15.4409x; 15.4409x over previous
import functools

import jax
import jax.numpy as jnp
from jax import lax
from jax.experimental import pallas as pl
from jax.experimental.pallas import tpu as pltpu
from jax.experimental.pallas import tpu_sc as plsc

N = 10000
D = 128
C = 128
G = 8

NCORES = 2
NSUB = 16
NW = NCORES * NSUB

N_PAD = 10240
BN = 640
NBLK = N_PAD // BN

E_RAW = 320000
E_TOT = E_RAW + N
E_PAD = 330752
CH = E_PAD // NW
GRP = 16
NG = CH // GRP

ROWS_PER_SUB = N_PAD // NSUB


def _mm_body(hin_ref, w_ref, as_ref, ad_ref, h_ref, asad_ref):
    h = jnp.dot(hin_ref[...], w_ref[...], preferred_element_type=jnp.float32)
    h_ref[...] = h
    asn = jnp.dot(h, as_ref[0, :], preferred_element_type=jnp.float32)
    adn = jnp.dot(h, ad_ref[0, :], preferred_element_type=jnp.float32)
    asad_ref[...] = jnp.stack([asn, adn])


def _tc_mm(hin, W, a_s, a_d):
    return pl.pallas_call(
        _mm_body,
        grid=(NBLK,),
        in_specs=[
            pl.BlockSpec((BN, D), lambda i: (i, 0)),
            pl.BlockSpec((D, C), lambda i: (0, 0)),
            pl.BlockSpec((1, C), lambda i: (0, 0)),
            pl.BlockSpec((1, C), lambda i: (0, 0)),
        ],
        out_specs=[
            pl.BlockSpec((BN, C), lambda i: (i, 0)),
            pl.BlockSpec((2, BN), lambda i: (0, i)),
        ],
        out_shape=[
            jax.ShapeDtypeStruct((N_PAD, C), jnp.float32),
            jax.ShapeDtypeStruct((2, N_PAD), jnp.float32),
        ],
    )(hin, W, a_s, a_d)


def _fin_mm_body(acc_ref, den_ref, b_ref, w_ref, as_ref, ad_ref, h_ref,
                 asad_ref):
    num = acc_ref[0] + acc_ref[1]
    den = den_ref[0, :] + den_ref[1, :] + 1e-16
    hprev = num / den[:, None] + b_ref[...]
    hprev = jnp.maximum(hprev, 0.0)
    h = jnp.dot(hprev, w_ref[...], preferred_element_type=jnp.float32)
    h_ref[...] = h
    asn = jnp.dot(h, as_ref[0, :], preferred_element_type=jnp.float32)
    adn = jnp.dot(h, ad_ref[0, :], preferred_element_type=jnp.float32)
    asad_ref[...] = jnp.stack([asn, adn])


def _tc_fin_mm(accs, dens, b_prev, W, a_s, a_d):
    return pl.pallas_call(
        _fin_mm_body,
        grid=(NBLK,),
        in_specs=[
            pl.BlockSpec((2, BN, C), lambda i: (0, i, 0)),
            pl.BlockSpec((2, BN), lambda i: (0, i)),
            pl.BlockSpec((1, C), lambda i: (0, 0)),
            pl.BlockSpec((D, C), lambda i: (0, 0)),
            pl.BlockSpec((1, C), lambda i: (0, 0)),
            pl.BlockSpec((1, C), lambda i: (0, 0)),
        ],
        out_specs=[
            pl.BlockSpec((BN, C), lambda i: (i, 0)),
            pl.BlockSpec((2, BN), lambda i: (0, i)),
        ],
        out_shape=[
            jax.ShapeDtypeStruct((N_PAD, C), jnp.float32),
            jax.ShapeDtypeStruct((2, N_PAD), jnp.float32),
        ],
    )(accs, dens, b_prev, W, a_s, a_d)


def _pool_body(acc_ref, den_ref, b_ref, batch_ref, fc1w_ref, fc1b_ref,
               fc2w_ref, fc2b_ref, out_ref, sums_ref, cnts_ref):
    i = pl.program_id(0)

    @pl.when(i == 0)
    def _():
        sums_ref[...] = jnp.zeros_like(sums_ref)
        cnts_ref[...] = jnp.zeros_like(cnts_ref)

    num = acc_ref[0] + acc_ref[1]
    den = den_ref[0, :] + den_ref[1, :] + 1e-16
    h3 = num / den[:, None] + b_ref[...]
    gids = lax.broadcasted_iota(jnp.int32, (G, BN), 0)
    oht = (batch_ref[0] == gids).astype(jnp.float32)
    sums_ref[...] += jnp.dot(oht, h3, preferred_element_type=jnp.float32)
    cnts_ref[...] += jnp.sum(oht, axis=1, keepdims=True)

    @pl.when(i == NBLK - 1)
    def _():
        pooled = sums_ref[...] / jnp.maximum(cnts_ref[...], 1.0)
        z = jnp.dot(pooled, fc1w_ref[...], preferred_element_type=jnp.float32)
        z = jnp.maximum(z + fc1b_ref[...], 0.0)
        out_ref[...] = (
            jnp.dot(z, fc2w_ref[...], preferred_element_type=jnp.float32)
            + fc2b_ref[...])


def _tc_pool(accs, dens, b3, batch_pad, fc1_W, fc1_b, fc2_Wp, fc2_bp):
    return pl.pallas_call(
        _pool_body,
        grid=(NBLK,),
        in_specs=[
            pl.BlockSpec((2, BN, C), lambda i: (0, i, 0)),
            pl.BlockSpec((2, BN), lambda i: (0, i)),
            pl.BlockSpec((1, C), lambda i: (0, 0)),
            pl.BlockSpec((1, 1, BN), lambda i: (i, 0, 0)),
            pl.BlockSpec((C, C), lambda i: (0, 0)),
            pl.BlockSpec((1, C), lambda i: (0, 0)),
            pl.BlockSpec((C, C), lambda i: (0, 0)),
            pl.BlockSpec((1, C), lambda i: (0, 0)),
        ],
        out_specs=pl.BlockSpec((G, C), lambda i: (0, 0)),
        out_shape=jax.ShapeDtypeStruct((G, C), jnp.float32),
        scratch_shapes=[
            pltpu.VMEM((G, C), jnp.float32),
            pltpu.VMEM((G, 1), jnp.float32),
        ],
    )(accs, dens, b3, batch_pad, fc1_W, fc1_b, fc2_Wp, fc2_bp)


def _sc_edge_body(h_hbm, asad_hbm, src_hbm, dst_hbm, accs_hbm, dens_hbm,
                  acc_sh, den_sh, asn_v, adn_v, src_v, dst_v, hbuf, exv,
                  zbuf, sem_g, sem_s, sem_d):
    c = lax.axis_index("c")
    s = lax.axis_index("s")
    wid = s * NCORES + c
    base = wid * CH

    pltpu.sync_copy(asad_hbm.at[0], asn_v)
    pltpu.sync_copy(asad_hbm.at[1], adn_v)
    pltpu.sync_copy(src_hbm.at[pl.ds(base, CH)], src_v)
    pltpu.sync_copy(dst_hbm.at[pl.ds(base, CH)], dst_v)

    def _zrow(i, carry):
        for j in range(C // 16):
            zbuf[i, pl.ds(j * 16, 16)] = jnp.zeros((16,), jnp.float32)
        return carry

    lax.fori_loop(0, 16, _zrow, 0)

    row0 = s * ROWS_PER_SUB

    def _zacc(i, carry):
        pltpu.sync_copy(zbuf, acc_sh.at[pl.ds(row0 + i * 16, 16), :])
        return carry

    lax.fori_loop(0, ROWS_PER_SUB // 16, _zacc, 0)
    for i in range(ROWS_PER_SUB // C):
        pltpu.sync_copy(zbuf.at[0], den_sh.at[pl.ds(row0 + i * C, C)])
    plsc.subcore_barrier()

    def _grp(g, carry):
        src16 = src_v[pl.ds(g * GRP, GRP)]
        dst16 = dst_v[pl.ds(g * GRP, GRP)]
        ea = plsc.load_gather(asn_v, [src16])
        eb = plsc.load_gather(adn_v, [dst16])
        e = ea + eb
        e = jnp.where(e >= 0.0, e, 0.2 * e)
        exv[...] = jnp.exp(e)
        pltpu.async_copy(h_hbm.at[src16], hbuf, sem_g).wait()
        for r in range(GRP):
            exr = plsc.load_gather(exv, [jnp.full((16,), r, jnp.int32)])
            for j in range(C // 16):
                hbuf[r, pl.ds(j * 16, 16)] = hbuf[r, pl.ds(j * 16, 16)] * exr
        pltpu.async_copy(hbuf, acc_sh.at[dst16], sem_s, add=True).wait()
        pltpu.async_copy(exv, den_sh.at[dst16], sem_d, add=True).wait()
        return carry

    lax.fori_loop(0, NG, _grp, 0)
    plsc.subcore_barrier()

    pltpu.sync_copy(acc_sh.at[pl.ds(row0, ROWS_PER_SUB), :],
                    accs_hbm.at[c, pl.ds(row0, ROWS_PER_SUB), :])
    pltpu.sync_copy(den_sh.at[pl.ds(row0, ROWS_PER_SUB)],
                    dens_hbm.at[c, pl.ds(row0, ROWS_PER_SUB)])


def _sc_edge(h, asad, src, dst):
    mesh = plsc.VectorSubcoreMesh(core_axis_name="c", subcore_axis_name="s",
                                  num_cores=NCORES, num_subcores=NSUB)
    return pl.kernel(
        _sc_edge_body,
        out_type=[
            jax.ShapeDtypeStruct((NCORES, N_PAD, C), jnp.float32),
            jax.ShapeDtypeStruct((NCORES, N_PAD), jnp.float32),
        ],
        mesh=mesh,
        compiler_params=pltpu.CompilerParams(needs_layout_passes=False),
        scratch_types=[
            pltpu.MemorySpace.VMEM_SHARED((N_PAD, C), jnp.float32),
            pltpu.MemorySpace.VMEM_SHARED((N_PAD,), jnp.float32),
            pltpu.VMEM((N_PAD,), jnp.float32),
            pltpu.VMEM((N_PAD,), jnp.float32),
            pltpu.VMEM((CH,), jnp.int32),
            pltpu.VMEM((CH,), jnp.int32),
            pltpu.VMEM((GRP, C), jnp.float32),
            pltpu.VMEM((GRP,), jnp.float32),
            pltpu.VMEM((16, C), jnp.float32),
            pltpu.SemaphoreType.DMA,
            pltpu.SemaphoreType.DMA,
            pltpu.SemaphoreType.DMA,
        ],
    )(h, asad, src, dst)


def kernel(x, edge_index, batch, W1, as1, ad1, b1, W2, as2, ad2, b2, W3, as3,
           ad3, b3, fc1_W, fc1_b, fc2_W, fc2_b):
    f32 = jnp.float32
    loops = jnp.arange(N, dtype=jnp.int32)
    pad_rows = (N + (jnp.arange(E_PAD - E_TOT, dtype=jnp.int32) %
                     (N_PAD - N))).astype(jnp.int32)
    src = jnp.concatenate([edge_index[0], loops, pad_rows])
    dst = jnp.concatenate([edge_index[1], loops, pad_rows])

    x_pad = jnp.concatenate([x, jnp.zeros((N_PAD - N, D), f32)])
    batch_pad = jnp.concatenate(
        [batch, jnp.full((N_PAD - N,), G, jnp.int32)]).reshape(NBLK, 1, BN)

    as1r, ad1r = as1.reshape(1, C), ad1.reshape(1, C)
    as2r, ad2r = as2.reshape(1, C), ad2.reshape(1, C)
    as3r, ad3r = as3.reshape(1, C), ad3.reshape(1, C)
    b1r, b2r, b3r = b1.reshape(1, C), b2.reshape(1, C), b3.reshape(1, C)
    fc1br = fc1_b.reshape(1, C)
    nc_out = fc2_W.shape[1]
    fc2_Wp = jnp.pad(fc2_W, ((0, 0), (0, C - nc_out)))
    fc2_bp = jnp.pad(fc2_b, (0, C - nc_out)).reshape(1, C)

    h1, asad1 = _tc_mm(x_pad, W1, as1r, ad1r)
    accs1, dens1 = _sc_edge(h1, asad1, src, dst)
    h2, asad2 = _tc_fin_mm(accs1, dens1, b1r, W2, as2r, ad2r)
    accs2, dens2 = _sc_edge(h2, asad2, src, dst)
    h3, asad3 = _tc_fin_mm(accs2, dens2, b2r, W3, as3r, ad3r)
    accs3, dens3 = _sc_edge(h3, asad3, src, dst)
    outp = _tc_pool(accs3, dens3, b3r, batch_pad, fc1_W, fc1br, fc2_Wp,
                    fc2_bp)
    return outp[:, :nc_out]


# speedup vs baseline: 21.8000x; 1.4118x over previous
import functools

import jax
import jax.numpy as jnp
from jax import lax
from jax.experimental import pallas as pl
from jax.experimental.pallas import tpu as pltpu
from jax.experimental.pallas import tpu_sc as plsc

N = 10000
D = 128
C = 128
G = 8

NCORES = 2
NSUB = 16
NW = NCORES * NSUB

N_PAD = 10240
BN = 640
NBLK = N_PAD // BN

E_RAW = 320000
E_TOT = E_RAW + N
E_PAD = 331776
CH = E_PAD // NW
GRP = 16
PIECE = CH // 4
NGP = PIECE // GRP
RING = 3

ROWS_PER_SUB = N_PAD // NSUB


def _mm_body(hin_ref, w_ref, as_ref, ad_ref, h_ref, asad_ref):
    h = jnp.dot(hin_ref[...], w_ref[...], preferred_element_type=jnp.float32)
    h_ref[...] = h
    asn = jnp.dot(h, as_ref[0, :], preferred_element_type=jnp.float32)
    adn = jnp.dot(h, ad_ref[0, :], preferred_element_type=jnp.float32)
    asad_ref[...] = jnp.stack([asn, adn])


def _tc_mm(hin, W, a_s, a_d):
    return pl.pallas_call(
        _mm_body,
        grid=(NBLK,),
        in_specs=[
            pl.BlockSpec((BN, D), lambda i: (i, 0)),
            pl.BlockSpec((D, C), lambda i: (0, 0)),
            pl.BlockSpec((1, C), lambda i: (0, 0)),
            pl.BlockSpec((1, C), lambda i: (0, 0)),
        ],
        out_specs=[
            pl.BlockSpec((BN, C), lambda i: (i, 0)),
            pl.BlockSpec((2, BN), lambda i: (0, i)),
        ],
        out_shape=[
            jax.ShapeDtypeStruct((N_PAD, C), jnp.float32),
            jax.ShapeDtypeStruct((2, N_PAD), jnp.float32),
        ],
    )(hin, W, a_s, a_d)


def _fin_mm_body(acc_ref, den_ref, b_ref, w_ref, as_ref, ad_ref, h_ref,
                 asad_ref):
    num = acc_ref[0] + acc_ref[1]
    den = den_ref[0, :] + den_ref[1, :] + 1e-16
    hprev = num / den[:, None] + b_ref[...]
    hprev = jnp.maximum(hprev, 0.0)
    h = jnp.dot(hprev, w_ref[...], preferred_element_type=jnp.float32)
    h_ref[...] = h
    asn = jnp.dot(h, as_ref[0, :], preferred_element_type=jnp.float32)
    adn = jnp.dot(h, ad_ref[0, :], preferred_element_type=jnp.float32)
    asad_ref[...] = jnp.stack([asn, adn])


def _tc_fin_mm(accs, dens, b_prev, W, a_s, a_d):
    return pl.pallas_call(
        _fin_mm_body,
        grid=(NBLK,),
        in_specs=[
            pl.BlockSpec((2, BN, C), lambda i: (0, i, 0)),
            pl.BlockSpec((2, BN), lambda i: (0, i)),
            pl.BlockSpec((1, C), lambda i: (0, 0)),
            pl.BlockSpec((D, C), lambda i: (0, 0)),
            pl.BlockSpec((1, C), lambda i: (0, 0)),
            pl.BlockSpec((1, C), lambda i: (0, 0)),
        ],
        out_specs=[
            pl.BlockSpec((BN, C), lambda i: (i, 0)),
            pl.BlockSpec((2, BN), lambda i: (0, i)),
        ],
        out_shape=[
            jax.ShapeDtypeStruct((N_PAD, C), jnp.float32),
            jax.ShapeDtypeStruct((2, N_PAD), jnp.float32),
        ],
    )(accs, dens, b_prev, W, a_s, a_d)


def _pool_body(acc_ref, den_ref, b_ref, batch_ref, fc1w_ref, fc1b_ref,
               fc2w_ref, fc2b_ref, out_ref, sums_ref, cnts_ref):
    i = pl.program_id(0)

    @pl.when(i == 0)
    def _():
        sums_ref[...] = jnp.zeros_like(sums_ref)
        cnts_ref[...] = jnp.zeros_like(cnts_ref)

    num = acc_ref[0] + acc_ref[1]
    den = den_ref[0, :] + den_ref[1, :] + 1e-16
    h3 = num / den[:, None] + b_ref[...]
    gids = lax.broadcasted_iota(jnp.int32, (G, BN), 0)
    oht = (batch_ref[0] == gids).astype(jnp.float32)
    sums_ref[...] += jnp.dot(oht, h3, preferred_element_type=jnp.float32)
    cnts_ref[...] += jnp.sum(oht, axis=1, keepdims=True)

    @pl.when(i == NBLK - 1)
    def _():
        pooled = sums_ref[...] / jnp.maximum(cnts_ref[...], 1.0)
        z = jnp.dot(pooled, fc1w_ref[...], preferred_element_type=jnp.float32)
        z = jnp.maximum(z + fc1b_ref[...], 0.0)
        out_ref[...] = (
            jnp.dot(z, fc2w_ref[...], preferred_element_type=jnp.float32)
            + fc2b_ref[...])


def _tc_pool(accs, dens, b3, batch_pad, fc1_W, fc1_b, fc2_Wp, fc2_bp):
    return pl.pallas_call(
        _pool_body,
        grid=(NBLK,),
        in_specs=[
            pl.BlockSpec((2, BN, C), lambda i: (0, i, 0)),
            pl.BlockSpec((2, BN), lambda i: (0, i)),
            pl.BlockSpec((1, C), lambda i: (0, 0)),
            pl.BlockSpec((1, 1, BN), lambda i: (i, 0, 0)),
            pl.BlockSpec((C, C), lambda i: (0, 0)),
            pl.BlockSpec((1, C), lambda i: (0, 0)),
            pl.BlockSpec((C, C), lambda i: (0, 0)),
            pl.BlockSpec((1, C), lambda i: (0, 0)),
        ],
        out_specs=pl.BlockSpec((G, C), lambda i: (0, 0)),
        out_shape=jax.ShapeDtypeStruct((G, C), jnp.float32),
        scratch_shapes=[
            pltpu.VMEM((G, C), jnp.float32),
            pltpu.VMEM((G, 1), jnp.float32),
        ],
    )(accs, dens, b3, batch_pad, fc1_W, fc1_b, fc2_Wp, fc2_bp)


def _sc_edge_body(h_hbm, asad_hbm, src_hbm, dst_hbm, accs_hbm, dens_hbm,
                  acc_sh, den_sh, asn_v, adn_v, src_v, dst_v, gbuf, sbuf,
                  exv, g0, g1, ssem, dsem):
    gsems = (g0, g1)
    c = lax.axis_index("c")
    s = lax.axis_index("s")
    wid = s * NCORES + c
    base = wid * CH

    pltpu.sync_copy(asad_hbm.at[0], asn_v)
    pltpu.sync_copy(asad_hbm.at[1], adn_v)

    def _zrow(i, carry):
        for j in range(C // 16):
            sbuf[i, pl.ds(j * 16, 16)] = jnp.zeros((16,), jnp.float32)
        return carry

    lax.fori_loop(0, GRP, _zrow, 0)

    row0 = s * ROWS_PER_SUB

    def _zacc(i, carry):
        pltpu.sync_copy(sbuf, acc_sh.at[pl.ds(row0 + i * GRP, GRP), :])
        return carry

    lax.fori_loop(0, ROWS_PER_SUB // GRP, _zacc, 0)
    for i in range(ROWS_PER_SUB // C):
        pltpu.sync_copy(sbuf.at[0], den_sh.at[pl.ds(row0 + i * C, C)])
    plsc.subcore_barrier()

    for p in range(CH // PIECE):
        pltpu.sync_copy(src_hbm.at[pl.ds(base + p * PIECE, PIECE)], src_v)
        pltpu.sync_copy(dst_hbm.at[pl.ds(base + p * PIECE, PIECE)], dst_v)
        src16p = src_v[pl.ds(0, GRP)]
        pltpu.async_copy(h_hbm.at[src16p], gbuf.at[0], gsems[0])

        def _iter(j, carry):
            for k in range(2):
                g = j * 2 + k
                src16 = src_v[pl.ds(g * GRP, GRP)]
                dst16 = dst_v[pl.ds(g * GRP, GRP)]
                pltpu.make_async_copy(h_hbm.at[src16], gbuf.at[k],
                                      gsems[k]).wait()
                gn = g + 1
                gn = jnp.where(gn >= NGP, gn - NGP, gn)
                srcn = src_v[pl.ds(gn * GRP, GRP)]
                pltpu.async_copy(h_hbm.at[srcn], gbuf.at[1 - k],
                                 gsems[1 - k])
                ea = plsc.load_gather(asn_v, [src16])
                eb = plsc.load_gather(adn_v, [dst16])
                e = ea + eb
                e = jnp.where(e >= 0.0, e, 0.2 * e)
                exv[...] = jnp.exp(e)
                for r in range(GRP):
                    exr = plsc.load_gather(
                        exv, [jnp.full((16,), r, jnp.int32)])
                    for jj in range(C // 16):
                        sbuf[r, pl.ds(jj * 16, 16)] = (
                            gbuf[k, r, pl.ds(jj * 16, 16)] * exr)
                pltpu.async_copy(sbuf, acc_sh.at[dst16], ssem,
                                 add=True).wait()
                pltpu.async_copy(exv, den_sh.at[dst16], dsem,
                                 add=True).wait()
            return carry

        lax.fori_loop(0, NGP // 2, _iter, 0)
        src16w = src_v[pl.ds(0, GRP)]
        pltpu.make_async_copy(h_hbm.at[src16w], gbuf.at[0], gsems[0]).wait()

    plsc.subcore_barrier()

    pltpu.sync_copy(acc_sh.at[pl.ds(row0, ROWS_PER_SUB), :],
                    accs_hbm.at[c, pl.ds(row0, ROWS_PER_SUB), :])
    pltpu.sync_copy(den_sh.at[pl.ds(row0, ROWS_PER_SUB)],
                    dens_hbm.at[c, pl.ds(row0, ROWS_PER_SUB)])


def _sc_edge(h, asad, src, dst):
    mesh = plsc.VectorSubcoreMesh(core_axis_name="c", subcore_axis_name="s",
                                  num_cores=NCORES, num_subcores=NSUB)
    return pl.kernel(
        _sc_edge_body,
        out_type=[
            jax.ShapeDtypeStruct((NCORES, N_PAD, C), jnp.float32),
            jax.ShapeDtypeStruct((NCORES, N_PAD), jnp.float32),
        ],
        mesh=mesh,
        compiler_params=pltpu.CompilerParams(needs_layout_passes=False),
        scratch_types=[
            pltpu.MemorySpace.VMEM_SHARED((N_PAD, C), jnp.float32),
            pltpu.MemorySpace.VMEM_SHARED((N_PAD,), jnp.float32),
            pltpu.VMEM((N_PAD,), jnp.float32),
            pltpu.VMEM((N_PAD,), jnp.float32),
            pltpu.VMEM((PIECE,), jnp.int32),
            pltpu.VMEM((PIECE,), jnp.int32),
            pltpu.VMEM((2, GRP, C), jnp.float32),
            pltpu.VMEM((GRP, C), jnp.float32),
            pltpu.VMEM((GRP,), jnp.float32),
            pltpu.SemaphoreType.DMA,
            pltpu.SemaphoreType.DMA,
            pltpu.SemaphoreType.DMA,
            pltpu.SemaphoreType.DMA,
        ],
    )(h, asad, src, dst)


def kernel(x, edge_index, batch, W1, as1, ad1, b1, W2, as2, ad2, b2, W3, as3,
           ad3, b3, fc1_W, fc1_b, fc2_W, fc2_b):
    f32 = jnp.float32
    loops = jnp.arange(N, dtype=jnp.int32)
    pad_rows = (N + (jnp.arange(E_PAD - E_TOT, dtype=jnp.int32) %
                     (N_PAD - N))).astype(jnp.int32)
    src = jnp.concatenate([edge_index[0], loops, pad_rows])
    dst = jnp.concatenate([edge_index[1], loops, pad_rows])

    x_pad = jnp.concatenate([x, jnp.zeros((N_PAD - N, D), f32)])
    batch_pad = jnp.concatenate(
        [batch, jnp.full((N_PAD - N,), G, jnp.int32)]).reshape(NBLK, 1, BN)

    as1r, ad1r = as1.reshape(1, C), ad1.reshape(1, C)
    as2r, ad2r = as2.reshape(1, C), ad2.reshape(1, C)
    as3r, ad3r = as3.reshape(1, C), ad3.reshape(1, C)
    b1r, b2r, b3r = b1.reshape(1, C), b2.reshape(1, C), b3.reshape(1, C)
    fc1br = fc1_b.reshape(1, C)
    nc_out = fc2_W.shape[1]
    fc2_Wp = jnp.pad(fc2_W, ((0, 0), (0, C - nc_out)))
    fc2_bp = jnp.pad(fc2_b, (0, C - nc_out)).reshape(1, C)

    h1, asad1 = _tc_mm(x_pad, W1, as1r, ad1r)
    accs1, dens1 = _sc_edge(h1, asad1, src, dst)
    h2, asad2 = _tc_fin_mm(accs1, dens1, b1r, W2, as2r, ad2r)
    accs2, dens2 = _sc_edge(h2, asad2, src, dst)
    h3, asad3 = _tc_fin_mm(accs2, dens2, b2r, W3, as3r, ad3r)
    accs3, dens3 = _sc_edge(h3, asad3, src, dst)
    outp = _tc_pool(accs3, dens3, b3r, batch_pad, fc1_W, fc1br, fc2_Wp,
                    fc2_bp)
    return outp[:, :nc_out]
